# Initial kernel scaffold; baseline (speedup 1.0000x reference)
import functools

import jax
import jax.numpy as jnp
from jax import lax
from jax.experimental import pallas as pl
from jax.experimental.pallas import tpu as pltpu
from jax.experimental.pallas import tpu_sc as plsc

N = 10000
E = 320000
D = 128

NC = 2
NS = 16
NW = NC * NS

NPAD = 10240
RPT = NPAD // NS
CH = 128
CPT = 79
E_PT = CPT * CH
EPAD = NW * E_PT

_MESH = plsc.VectorSubcoreMesh(core_axis_name="c", subcore_axis_name="s")


@functools.partial(
    pl.kernel,
    out_type=jax.ShapeDtypeStruct((NC, NPAD), jnp.float32),
    mesh=_MESH,
    scratch_types=[
        pltpu.VMEM((E_PT,), jnp.int32),
        pltpu.VMEM((NPAD,), jnp.float32),
        pltpu.VMEM_SHARED((NPAD,), jnp.float32),
    ],
)
def _sc_degree(src_hbm, hist_out, idx_v, hist_v, hist_sh):
    c = lax.axis_index("c")
    s = lax.axis_index("s")
    wid = c * NS + s
    pltpu.sync_copy(src_hbm.at[pl.ds(wid * E_PT, E_PT)], idx_v)

    zeros16 = jnp.zeros((16,), jnp.float32)

    def _zero(i, carry):
        hist_v[pl.ds(i * 16, 16)] = zeros16
        return carry

    lax.fori_loop(0, NPAD // 16, _zero, 0)

    ones16 = jnp.ones((16,), jnp.float32)

    def _count(i, carry):
        ix = idx_v[pl.ds(i * 16, 16)]
        plsc.addupdate_scatter(hist_v, [ix], ones16)
        return carry

    lax.fori_loop(0, E_PT // 16, _count, 0)

    @pl.when(s == 0)
    def _():
        pltpu.sync_copy(hist_v, hist_sh)

    plsc.subcore_barrier()

    @pl.when(s != 0)
    def _():
        pltpu.sync_copy(hist_v, hist_sh, add=True)

    plsc.subcore_barrier()
    pltpu.sync_copy(
        hist_sh.at[pl.ds(s * RPT, RPT)], hist_out.at[c, pl.ds(s * RPT, RPT)]
    )


def _prescale_body(x_ref, w_ref, hist_ref, hs_ref, norm_ref):
    deg = hist_ref[:, 0:1] + hist_ref[:, 1:2] + 1.0
    nrm = lax.rsqrt(deg)
    norm_ref[...] = nrm
    h = jnp.dot(x_ref[...], w_ref[...], preferred_element_type=jnp.float32)
    hs_ref[...] = h * nrm


_PRE_BLK = 256


def _prescale(x_p, w, hist_t):
    return pl.pallas_call(
        _prescale_body,
        grid=(NPAD // _PRE_BLK,),
        in_specs=[
            pl.BlockSpec((_PRE_BLK, D), lambda i: (i, 0)),
            pl.BlockSpec((D, D), lambda i: (0, 0)),
            pl.BlockSpec((_PRE_BLK, NC), lambda i: (i, 0)),
        ],
        out_specs=[
            pl.BlockSpec((_PRE_BLK, D), lambda i: (i, 0)),
            pl.BlockSpec((_PRE_BLK, 1), lambda i: (i, 0)),
        ],
        out_shape=[
            jax.ShapeDtypeStruct((NPAD, D), jnp.float32),
            jax.ShapeDtypeStruct((NPAD, 1), jnp.float32),
        ],
    )(x_p, w, hist_t)


@functools.partial(
    pl.kernel,
    out_type=jax.ShapeDtypeStruct((NC, NPAD, D), jnp.float32),
    mesh=_MESH,
    scratch_types=[
        pltpu.VMEM((CPT, CH), jnp.int32),
        pltpu.VMEM((CPT, CH), jnp.int32),
        pltpu.VMEM((CH, D), jnp.float32),
        pltpu.VMEM_SHARED((NPAD, D), jnp.float32),
        pltpu.SemaphoreType.DMA,
    ],
)
def _sc_aggregate(hs_hbm, src_hbm, dst_hbm, out_hbm, sidx, didx, rows, agg_sh, sem):
    c = lax.axis_index("c")
    s = lax.axis_index("s")
    wid = c * NS + s
    base = s * RPT

    @pl.when(c == 0)
    def _():
        pltpu.sync_copy(hs_hbm.at[pl.ds(base, RPT)], agg_sh.at[pl.ds(base, RPT)])

    @pl.when(c != 0)
    def _():
        for k in range(4):
            pltpu.sync_copy(
                hs_hbm.at[pl.ds(N, 160)], agg_sh.at[pl.ds(base + k * 160, 160)]
            )

    pltpu.sync_copy(src_hbm.at[pl.ds(wid * CPT, CPT)], sidx)
    pltpu.sync_copy(dst_hbm.at[pl.ds(wid * CPT, CPT)], didx)
    plsc.subcore_barrier()

    def _chunk(j, carry):
        pltpu.async_copy(hs_hbm.at[sidx.at[j]], rows, sem).wait()
        pltpu.sync_copy(rows, agg_sh.at[didx.at[j]], add=True)
        return carry

    lax.fori_loop(0, CPT, _chunk, 0)

    plsc.subcore_barrier()
    pltpu.sync_copy(
        agg_sh.at[pl.ds(base, RPT)], out_hbm.at[c, pl.ds(base, RPT)]
    )


def _final_body(a_ref, b_ref, norm_ref, out_ref):
    r = (a_ref[...] + b_ref[...]) * norm_ref[...]
    ss = jnp.sum(r * r, axis=1, keepdims=True)
    denom = jnp.maximum(jnp.sqrt(ss), 1e-12)
    out_ref[...] = jnp.tanh(r / denom)


_FIN_BLK = 400


def _final(a, b, norm):
    return pl.pallas_call(
        _final_body,
        grid=(N // _FIN_BLK,),
        in_specs=[
            pl.BlockSpec((_FIN_BLK, D), lambda i: (i, 0)),
            pl.BlockSpec((_FIN_BLK, D), lambda i: (i, 0)),
            pl.BlockSpec((_FIN_BLK, 1), lambda i: (i, 0)),
        ],
        out_specs=pl.BlockSpec((_FIN_BLK, D), lambda i: (i, 0)),
        out_shape=jax.ShapeDtypeStruct((N, D), jnp.float32),
    )(a, b, norm)


@jax.jit
def kernel(x, edge_index, W):
    src = edge_index[0]
    dst = edge_index[1]
    pad = jnp.full((EPAD - E,), N, jnp.int32)
    src_p = jnp.concatenate([src, pad])
    dst_p = jnp.concatenate([dst, pad])

    x_p = jnp.zeros((NPAD, D), jnp.float32).at[:N].set(x)

    hist = _sc_degree(src_p)
    hs, norm = _prescale(x_p, W, hist.T)
    partials = _sc_aggregate(
        hs, src_p.reshape(NW * CPT, CH), dst_p.reshape(NW * CPT, CH)
    )
    return _final(partials[0], partials[1], norm[:N])


# speedup vs baseline: 8.0371x; 8.0371x over previous
import functools

import jax
import jax.numpy as jnp
from jax import lax
from jax.experimental import pallas as pl
from jax.experimental.pallas import tpu as pltpu
from jax.experimental.pallas import tpu_sc as plsc

N = 10000
E = 320000
D = 128

NC = 2
NS = 16
NW = NC * NS

NPAD = 10240
RPT = NPAD // NS
CH = 128
CPT = 80
E_PT = CPT * CH
EPAD = NW * E_PT

_MESH = plsc.VectorSubcoreMesh(core_axis_name="c", subcore_axis_name="s")


@functools.partial(
    pl.kernel,
    out_type=jax.ShapeDtypeStruct((NW, NPAD), jnp.float32),
    mesh=_MESH,
    compiler_params=pltpu.CompilerParams(needs_layout_passes=False),
    scratch_types=[
        pltpu.VMEM((E_PT,), jnp.int32),
        pltpu.VMEM((NPAD,), jnp.float32),
    ],
)
def _sc_degree(src_hbm, hist_out, idx_v, hist_v):
    c = lax.axis_index("c")
    s = lax.axis_index("s")
    wid = c * NS + s
    pltpu.sync_copy(src_hbm.at[pl.ds(wid * E_PT, E_PT)], idx_v)

    zeros16 = jnp.zeros((16,), jnp.float32)

    def _zero(i, carry):
        hist_v[pl.ds(i * 16, 16)] = zeros16
        return carry

    lax.fori_loop(0, NPAD // 16, _zero, 0)

    ones16 = jnp.ones((16,), jnp.float32)

    def _count(i, carry):
        ix = idx_v[pl.ds(i * 16, 16)]
        plsc.addupdate_scatter(hist_v, [ix], ones16)
        return carry

    lax.fori_loop(0, E_PT // 16, _count, 0)

    pltpu.sync_copy(hist_v, hist_out.at[wid])


def _prescale_body(x_ref, w_ref, hist_ref, hs_ref, norm_ref):
    deg = jnp.sum(hist_ref[...], axis=1, keepdims=True) + 1.0
    nrm = lax.rsqrt(deg)
    norm_ref[...] = nrm
    h = jnp.dot(x_ref[...], w_ref[...], preferred_element_type=jnp.float32)
    hs_ref[...] = h * nrm


_PRE_BLK = 256


def _prescale(x_p, w, hist_t):
    return pl.pallas_call(
        _prescale_body,
        grid=(NPAD // _PRE_BLK,),
        in_specs=[
            pl.BlockSpec((_PRE_BLK, D), lambda i: (i, 0)),
            pl.BlockSpec((D, D), lambda i: (0, 0)),
            pl.BlockSpec((_PRE_BLK, NW), lambda i: (i, 0)),
        ],
        out_specs=[
            pl.BlockSpec((_PRE_BLK, D), lambda i: (i, 0)),
            pl.BlockSpec((_PRE_BLK, 1), lambda i: (i, 0)),
        ],
        out_shape=[
            jax.ShapeDtypeStruct((NPAD, D), jnp.float32),
            jax.ShapeDtypeStruct((NPAD, 1), jnp.float32),
        ],
    )(x_p, w, hist_t)


@functools.partial(
    pl.kernel,
    out_type=jax.ShapeDtypeStruct((NC, NPAD, D), jnp.float32),
    mesh=_MESH,
    compiler_params=pltpu.CompilerParams(needs_layout_passes=False),
    scratch_types=[
        pltpu.VMEM((CPT, CH), jnp.int32),
        pltpu.VMEM((CPT, CH), jnp.int32),
        pltpu.VMEM((CH, D), jnp.float32),
        pltpu.VMEM_SHARED((NPAD, D), jnp.float32),
        pltpu.SemaphoreType.DMA,
    ],
)
def _sc_aggregate(hs_hbm, src_hbm, dst_hbm, out_hbm, sidx, didx, rows, agg_sh, sem):
    c = lax.axis_index("c")
    s = lax.axis_index("s")
    wid = c * NS + s
    base = s * RPT

    @pl.when(c == 0)
    def _():
        pltpu.sync_copy(hs_hbm.at[pl.ds(base, RPT)], agg_sh.at[pl.ds(base, RPT)])

    @pl.when(c != 0)
    def _():
        for k in range(4):
            pltpu.sync_copy(
                hs_hbm.at[pl.ds(N, 160)], agg_sh.at[pl.ds(base + k * 160, 160)]
            )

    pltpu.sync_copy(src_hbm.at[pl.ds(wid * CPT, CPT)], sidx)
    pltpu.sync_copy(dst_hbm.at[pl.ds(wid * CPT, CPT)], didx)
    plsc.subcore_barrier()

    def _chunk(j, carry):
        pltpu.async_copy(hs_hbm.at[sidx.at[j]], rows, sem).wait()
        pltpu.sync_copy(rows, agg_sh.at[didx.at[j]], add=True)
        return carry

    lax.fori_loop(0, CPT, _chunk, 0)

    plsc.subcore_barrier()
    pltpu.sync_copy(
        agg_sh.at[pl.ds(base, RPT)], out_hbm.at[c, pl.ds(base, RPT)]
    )


def _final_body(a_ref, b_ref, norm_ref, out_ref):
    r = (a_ref[...] + b_ref[...]) * norm_ref[...]
    ss = jnp.sum(r * r, axis=1, keepdims=True)
    denom = jnp.maximum(jnp.sqrt(ss), 1e-12)
    out_ref[...] = jnp.tanh(r / denom)


_FIN_BLK = 400


def _final(a, b, norm):
    return pl.pallas_call(
        _final_body,
        grid=(N // _FIN_BLK,),
        in_specs=[
            pl.BlockSpec((_FIN_BLK, D), lambda i: (i, 0)),
            pl.BlockSpec((_FIN_BLK, D), lambda i: (i, 0)),
            pl.BlockSpec((_FIN_BLK, 1), lambda i: (i, 0)),
        ],
        out_specs=pl.BlockSpec((_FIN_BLK, D), lambda i: (i, 0)),
        out_shape=jax.ShapeDtypeStruct((N, D), jnp.float32),
    )(a, b, norm)


@jax.jit
def kernel(x, edge_index, W):
    src = edge_index[0]
    dst = edge_index[1]
    pad = jnp.full((EPAD - E,), N, jnp.int32)
    src_p = jnp.concatenate([src, pad])
    dst_p = jnp.concatenate([dst, pad])

    x_p = jnp.zeros((NPAD, D), jnp.float32).at[:N].set(x)

    hist = _sc_degree(src_p)
    hs, norm = _prescale(x_p, W, hist.T)
    partials = _sc_aggregate(
        hs, src_p.reshape(NW * CPT, CH), dst_p.reshape(NW * CPT, CH)
    )
    return _final(partials[0], partials[1], norm[:N])


# speedup vs baseline: 25.0938x; 3.1223x over previous
import functools

import jax
import jax.numpy as jnp
from jax import lax
from jax.experimental import pallas as pl
from jax.experimental.pallas import tpu as pltpu
from jax.experimental.pallas import tpu_sc as plsc

N = 10000
E = 320000
D = 128

NC = 2
NS = 16
NW = NC * NS

NPAD = 10112
RPT = NPAD // NS
CH = 128
CPT = 80
E_PT = CPT * CH
EPAD = NW * E_PT

_MESH = plsc.VectorSubcoreMesh(core_axis_name="c", subcore_axis_name="s")


@functools.partial(
    pl.kernel,
    out_type=jax.ShapeDtypeStruct((NW, NPAD), jnp.float32),
    mesh=_MESH,
    compiler_params=pltpu.CompilerParams(needs_layout_passes=False),
    scratch_types=[
        pltpu.VMEM((E_PT,), jnp.int32),
        pltpu.VMEM((NPAD,), jnp.float32),
    ],
)
def _sc_degree(src_hbm, hist_out, idx_v, hist_v):
    c = lax.axis_index("c")
    s = lax.axis_index("s")
    wid = c * NS + s
    pltpu.sync_copy(src_hbm.at[pl.ds(wid * E_PT, E_PT)], idx_v)

    zeros16 = jnp.zeros((16,), jnp.float32)

    def _zero(i, carry):
        hist_v[pl.ds(i * 16, 16)] = zeros16
        return carry

    lax.fori_loop(0, NPAD // 16, _zero, 0)

    ones16 = jnp.ones((16,), jnp.float32)

    def _count(i, carry):
        ix = idx_v[pl.ds(i * 16, 16)]
        plsc.addupdate_scatter(hist_v, [ix], ones16)
        return carry

    lax.fori_loop(0, E_PT // 16, _count, 0)

    pltpu.sync_copy(hist_v, hist_out.at[wid])


def _prescale_body(x_ref, w_ref, hist_ref, hs_ref, norm_ref):
    deg = jnp.sum(hist_ref[...], axis=1, keepdims=True) + 1.0
    nrm = lax.rsqrt(deg)
    norm_ref[...] = nrm
    h = jnp.dot(x_ref[...], w_ref[...], preferred_element_type=jnp.float32)
    hs_ref[...] = h * nrm


_PRE_BLK = 632


def _prescale(x_p, w, hist_t):
    return pl.pallas_call(
        _prescale_body,
        grid=(NPAD // _PRE_BLK,),
        in_specs=[
            pl.BlockSpec((_PRE_BLK, D), lambda i: (i, 0)),
            pl.BlockSpec((D, D), lambda i: (0, 0)),
            pl.BlockSpec((_PRE_BLK, NW), lambda i: (i, 0)),
        ],
        out_specs=[
            pl.BlockSpec((_PRE_BLK, D), lambda i: (i, 0)),
            pl.BlockSpec((_PRE_BLK, 1), lambda i: (i, 0)),
        ],
        out_shape=[
            jax.ShapeDtypeStruct((NPAD, D), jnp.float32),
            jax.ShapeDtypeStruct((NPAD, 1), jnp.float32),
        ],
    )(x_p, w, hist_t)


@functools.partial(
    pl.kernel,
    out_type=jax.ShapeDtypeStruct((NC, NPAD, D), jnp.float32),
    mesh=_MESH,
    compiler_params=pltpu.CompilerParams(needs_layout_passes=False),
    scratch_types=[
        pltpu.VMEM((CPT // 2, CH), jnp.int32),
        pltpu.VMEM((CPT // 2, CH), jnp.int32),
        [pltpu.VMEM((CH, D), jnp.float32)] * 2,
        pltpu.VMEM_SHARED((NPAD, D), jnp.float32),
        [pltpu.SemaphoreType.DMA] * 2,
        [pltpu.SemaphoreType.DMA] * 2,
    ],
)
def _sc_aggregate(
    hs_hbm, src_hbm, dst_hbm, out_hbm, sidx, didx, rows, agg_sh, gsems, ssems
):
    c = lax.axis_index("c")
    s = lax.axis_index("s")
    wid = c * NS + s
    base = s * RPT

    @pl.when(c == 0)
    def _():
        pltpu.sync_copy(hs_hbm.at[pl.ds(base, RPT)], agg_sh.at[pl.ds(base, RPT)])

    @pl.when(c != 0)
    def _():
        for k in range(5):
            pltpu.sync_copy(
                hs_hbm.at[pl.ds(N, 112)], agg_sh.at[pl.ds(base + k * 112, 112)]
            )
        pltpu.sync_copy(
            hs_hbm.at[pl.ds(N, 72)], agg_sh.at[pl.ds(base + 560, 72)]
        )

    plsc.subcore_barrier()

    CPP = CPT // 2
    for p in range(2):
        pltpu.sync_copy(src_hbm.at[pl.ds(wid * CPT + p * CPP, CPP)], sidx)
        pltpu.sync_copy(dst_hbm.at[pl.ds(wid * CPT + p * CPP, CPP)], didx)

        for t in range(2):
            pltpu.async_copy(hs_hbm.at[sidx.at[t]], rows[t], gsems[t])

        def _group(i, carry):
            for t in range(2):
                j = i * 2 + t
                pltpu.make_async_copy(
                    hs_hbm.at[sidx.at[j]], rows[t], gsems[t]
                ).wait()
                pltpu.async_copy(rows[t], agg_sh.at[didx.at[j]], ssems[t], add=True)

                @pl.when(i < CPP // 2 - 1)
                def _():
                    pltpu.make_async_copy(
                        rows[t], agg_sh.at[didx.at[j]], ssems[t]
                    ).wait()
                    pltpu.async_copy(hs_hbm.at[sidx.at[j + 2]], rows[t], gsems[t])

            return carry

        lax.fori_loop(0, CPP // 2, _group, 0)

        for t in range(2):
            pltpu.make_async_copy(
                rows[t], agg_sh.at[didx.at[CPP - 2 + t]], ssems[t]
            ).wait()

    plsc.subcore_barrier()
    pltpu.sync_copy(
        agg_sh.at[pl.ds(base, RPT)], out_hbm.at[c, pl.ds(base, RPT)]
    )


def _final_body(a_ref, b_ref, norm_ref, out_ref):
    r = (a_ref[...] + b_ref[...]) * norm_ref[...]
    ss = jnp.sum(r * r, axis=1, keepdims=True)
    denom = jnp.maximum(jnp.sqrt(ss), 1e-12)
    out_ref[...] = jnp.tanh(r / denom)


_FIN_BLK = 400


def _final(a, b, norm):
    return pl.pallas_call(
        _final_body,
        grid=(N // _FIN_BLK,),
        in_specs=[
            pl.BlockSpec((_FIN_BLK, D), lambda i: (i, 0)),
            pl.BlockSpec((_FIN_BLK, D), lambda i: (i, 0)),
            pl.BlockSpec((_FIN_BLK, 1), lambda i: (i, 0)),
        ],
        out_specs=pl.BlockSpec((_FIN_BLK, D), lambda i: (i, 0)),
        out_shape=jax.ShapeDtypeStruct((N, D), jnp.float32),
    )(a, b, norm)


@jax.jit
def kernel(x, edge_index, W):
    src = edge_index[0]
    dst = edge_index[1]
    pad = N + jnp.arange(EPAD - E, dtype=jnp.int32) % (NPAD - N)
    src_p = jnp.concatenate([src, pad])
    dst_p = jnp.concatenate([dst, pad])

    x_p = jnp.zeros((NPAD, D), jnp.float32).at[:N].set(x)

    hist = _sc_degree(src_p)
    hs, norm = _prescale(x_p, W, hist.T)
    partials = _sc_aggregate(
        hs, src_p.reshape(NW * CPT, CH), dst_p.reshape(NW * CPT, CH)
    )
    return _final(partials[0], partials[1], norm[:N])
